# baseline (speedup 1.0000x reference)
import functools

import jax
import jax.numpy as jnp
from jax import lax
from jax.experimental import pallas as pl
from jax.experimental.pallas import tpu as pltpu
from jax.experimental.pallas import tpu_sc as plsc

D_MODEL = 64
CHUNK = 128


def _num_workers():
    try:
        info = plsc.get_sparse_core_info()
        return int(info.num_cores), int(info.num_subcores)
    except Exception:
        return 2, 16


@functools.lru_cache(maxsize=None)
def _build(B: int):
    nc, ns = _num_workers()
    nw = nc * ns
    assert B % (nw * CHUNK) == 0
    b_per_w = B // nw
    n_chunks = b_per_w // CHUNK

    mesh = plsc.VectorSubcoreMesh(core_axis_name="c", subcore_axis_name="s")

    @functools.partial(
        pl.kernel,
        out_type=jax.ShapeDtypeStruct((B, D_MODEL), jnp.float32),
        mesh=mesh,
        scratch_types=[
            pltpu.VMEM((n_chunks, CHUNK), jnp.int32),
            pltpu.VMEM((2, CHUNK, D_MODEL), jnp.float32),
            pltpu.SemaphoreType.DMA,
        ],
        compiler_params=pltpu.CompilerParams(use_tc_tiling_on_sc=False),
    )
    def embed(idx_hbm, table_hbm, out_hbm, idx_v, rows_v, sem):
        wid = lax.axis_index("s") * nc + lax.axis_index("c")
        base = wid * b_per_w
        pltpu.sync_copy(idx_hbm.at[wid], idx_v)

        def body(i, carry):
            pltpu.async_copy(
                table_hbm.at[idx_v.at[i]], rows_v.at[0], sem
            ).wait()
            pltpu.sync_copy(
                rows_v.at[0], out_hbm.at[pl.ds(base + i * CHUNK, CHUNK)]
            )
            return carry

        lax.fori_loop(0, n_chunks, body, 0)

    return embed, nw, n_chunks


@jax.jit
def kernel(x, table):
    B = x.size
    embed, nw, n_chunks = _build(B)
    idx = x.reshape(nw, n_chunks, CHUNK)
    out = embed(idx, table)
    return out.reshape(x.shape + (D_MODEL,))


# speedup vs baseline: 1.1144x; 1.1144x over previous
import functools

import jax
import jax.numpy as jnp
from jax import lax
from jax.experimental import pallas as pl
from jax.experimental.pallas import tpu as pltpu
from jax.experimental.pallas import tpu_sc as plsc

D_MODEL = 64
CHUNK = 128
GRP = 4
ROWS_PER_GRP = GRP * CHUNK


def _num_workers():
    try:
        info = plsc.get_sparse_core_info()
        return int(info.num_cores), int(info.num_subcores)
    except Exception:
        return 2, 16


@functools.lru_cache(maxsize=None)
def _build(B: int):
    nc, ns = _num_workers()
    nw = nc * ns
    assert B % (nw * CHUNK * GRP * 2) == 0
    b_per_w = B // nw
    n_chunks = b_per_w // CHUNK
    n_rounds = n_chunks // GRP
    n_pairs = n_rounds // 2

    mesh = plsc.VectorSubcoreMesh(core_axis_name="c", subcore_axis_name="s")

    @functools.partial(
        pl.kernel,
        out_type=jax.ShapeDtypeStruct((B, D_MODEL), jnp.float32),
        mesh=mesh,
        scratch_types=[
            pltpu.VMEM((n_chunks, CHUNK), jnp.int32),
            pltpu.VMEM((2, ROWS_PER_GRP, D_MODEL), jnp.float32),
            pltpu.SemaphoreType.DMA,
            pltpu.SemaphoreType.DMA,
            pltpu.SemaphoreType.DMA,
            pltpu.SemaphoreType.DMA,
        ],
        compiler_params=pltpu.CompilerParams(use_tc_tiling_on_sc=False),
    )
    def embed(idx_hbm, table_hbm, out_hbm, idx_v, rows_v, g0, g1, s0, s1):
        wid = lax.axis_index("s") * nc + lax.axis_index("c")
        base = wid * b_per_w
        gsem = (g0, g1)
        ssem = (s0, s1)
        pltpu.sync_copy(idx_hbm.at[wid], idx_v)

        def fire_group(p, grp):
            for b in range(GRP):
                pltpu.async_copy(
                    table_hbm.at[idx_v.at[p * GRP + b]],
                    rows_v.at[grp, pl.ds(b * CHUNK, CHUNK)],
                    gsem[grp],
                )

        def drain_group(p, grp):
            for b in range(GRP):
                pltpu.make_async_copy(
                    table_hbm.at[idx_v.at[p * GRP + b]],
                    rows_v.at[grp, pl.ds(b * CHUNK, CHUNK)],
                    gsem[grp],
                ).wait()

        def out_slab(p):
            return out_hbm.at[pl.ds(base + p * ROWS_PER_GRP, ROWS_PER_GRP)]

        fire_group(0, 0)
        fire_group(1, 1)

        def body(q, carry):
            for grp in range(2):
                p = 2 * q + grp
                drain_group(p, grp)
                pltpu.async_copy(rows_v.at[grp], out_slab(p), ssem[grp])
                pltpu.make_async_copy(
                    rows_v.at[grp], out_slab(p), ssem[grp]
                ).wait()
                fire_group(p + 2, grp)
            return carry

        lax.fori_loop(0, n_pairs - 1, body, 0)

        for grp in range(2):
            p = n_rounds - 2 + grp
            drain_group(p, grp)
            pltpu.async_copy(rows_v.at[grp], out_slab(p), ssem[grp])
        for grp in range(2):
            p = n_rounds - 2 + grp
            pltpu.make_async_copy(rows_v.at[grp], out_slab(p), ssem[grp]).wait()

    return embed, nw, n_chunks


@jax.jit
def kernel(x, table):
    B = x.size
    embed, nw, n_chunks = _build(B)
    idx = x.reshape(nw, n_chunks, CHUNK)
    out = embed(idx, table)
    return out.reshape(x.shape + (D_MODEL,))


# speedup vs baseline: 1.1167x; 1.0021x over previous
import functools

import jax
import jax.numpy as jnp
from jax import lax
from jax.experimental import pallas as pl
from jax.experimental.pallas import tpu as pltpu
from jax.experimental.pallas import tpu_sc as plsc

D_MODEL = 64
SPLITS = ((0, 104), (104, 96))
ROWS_PER_ROUND = 2


def _num_workers():
    try:
        info = plsc.get_sparse_core_info()
        return int(info.num_cores), int(info.num_subcores)
    except Exception:
        return 2, 16


@functools.lru_cache(maxsize=None)
def _build(n_rows: int, n_cols: int):
    nc, ns = _num_workers()
    nw = nc * ns
    assert n_rows % (nw * ROWS_PER_ROUND * 2) == 0
    rows_per_w = n_rows // nw
    n_rounds = rows_per_w // ROWS_PER_ROUND
    n_pairs = n_rounds // 2

    mesh = plsc.VectorSubcoreMesh(core_axis_name="c", subcore_axis_name="s")

    @functools.partial(
        pl.kernel,
        out_type=jax.ShapeDtypeStruct((n_rows, n_cols, D_MODEL), jnp.float32),
        mesh=mesh,
        scratch_types=[
            pltpu.VMEM((rows_per_w, n_cols), jnp.int32),
            pltpu.VMEM((2, ROWS_PER_ROUND, n_cols, D_MODEL), jnp.float32),
            pltpu.SemaphoreType.DMA,
            pltpu.SemaphoreType.DMA,
            pltpu.SemaphoreType.DMA,
            pltpu.SemaphoreType.DMA,
        ],
        compiler_params=pltpu.CompilerParams(use_tc_tiling_on_sc=False),
    )
    def embed(idx_hbm, table_hbm, out_hbm, idx_v, rows_v, g0, g1, s0, s1):
        wid = lax.axis_index("s") * nc + lax.axis_index("c")
        row0 = wid * rows_per_w
        gsem = (g0, g1)
        ssem = (s0, s1)
        pltpu.sync_copy(idx_hbm.at[pl.ds(row0, rows_per_w)], idx_v)

        def transfers(p, grp):
            out = []
            for rr in range(ROWS_PER_ROUND):
                r = p * ROWS_PER_ROUND + rr
                for off, sz in SPLITS:
                    out.append((
                        table_hbm.at[idx_v.at[r, pl.ds(off, sz)]],
                        rows_v.at[grp, rr, pl.ds(off, sz)],
                    ))
            return out

        def fire_group(p, grp):
            for src, dst in transfers(p, grp):
                pltpu.async_copy(src, dst, gsem[grp])

        def drain_group(p, grp):
            for src, dst in transfers(p, grp):
                pltpu.make_async_copy(src, dst, gsem[grp]).wait()

        def out_slab(p):
            return out_hbm.at[pl.ds(row0 + p * ROWS_PER_ROUND, ROWS_PER_ROUND)]

        fire_group(0, 0)
        fire_group(1, 1)

        def body(q, carry):
            for grp in range(2):
                p = 2 * q + grp
                drain_group(p, grp)
                pltpu.async_copy(rows_v.at[grp], out_slab(p), ssem[grp])
                pltpu.make_async_copy(
                    rows_v.at[grp], out_slab(p), ssem[grp]
                ).wait()
                fire_group(p + 2, grp)
            return carry

        lax.fori_loop(0, n_pairs - 1, body, 0)

        for grp in range(2):
            p = n_rounds - 2 + grp
            drain_group(p, grp)
            pltpu.async_copy(rows_v.at[grp], out_slab(p), ssem[grp])
        for grp in range(2):
            p = n_rounds - 2 + grp
            pltpu.make_async_copy(rows_v.at[grp], out_slab(p), ssem[grp]).wait()

    return embed


@jax.jit
def kernel(x, table):
    embed = _build(*x.shape)
    return embed(x, table)
